# baseline (speedup 1.0000x reference)
import functools

import jax
import jax.numpy as jnp
from jax import lax
from jax.experimental import pallas as pl
from jax.experimental.pallas import tpu as pltpu
from jax.experimental.pallas import tpu_sc as plsc

_K = 8192
_D = 256
_N = 8192
_COMMIT = 0.25

_BN = 512
_NI = _N // _BN
_CHUNKS = (0, 2048, 4096, 6144, 8192)


def _argmin_body(z_ref, z2_ref, e_ref, e2_ref, idx_ref):
    zb = z_ref[...].astype(jnp.bfloat16)
    z2 = z2_ref[...]
    acc_v = None
    acc_i = None
    for c in range(len(_CHUNKS) - 1):
        base = _CHUNKS[c]
        width = _CHUNKS[c + 1] - base
        eb = e_ref[pl.ds(base, width), :].astype(jnp.bfloat16)
        ze = lax.dot_general(
            zb, eb,
            dimension_numbers=(((1,), (1,)), ((), ())),
            preferred_element_type=jnp.float32,
        )
        dist = (z2 + e2_ref[:, pl.ds(base, width)]) - 2.0 * ze
        cmin = jnp.min(dist, axis=1, keepdims=True)
        gidx = base + lax.broadcasted_iota(jnp.int32, (_BN, width), 1)
        cidx = jnp.min(jnp.where(dist == cmin, gidx, _K), axis=1, keepdims=True)
        if c == 0:
            acc_v, acc_i = cmin, cidx
        else:
            acc_rounded = acc_v.astype(jnp.bfloat16).astype(jnp.float32)
            upd = cmin < acc_rounded
            acc_v = jnp.where(upd, cmin, acc_v)
            acc_i = jnp.where(upd, cidx, acc_i)
    idx_ref[...] = acc_i.reshape(1, _BN, 1)


def _argmin_indices(z, z2, e2, embedding):
    out = pl.pallas_call(
        _argmin_body,
        grid=(_NI,),
        in_specs=[
            pl.BlockSpec((_BN, _D), lambda ni: (ni, 0)),
            pl.BlockSpec((_BN, 1), lambda ni: (ni, 0)),
            pl.BlockSpec((_K, _D), lambda ni: (0, 0)),
            pl.BlockSpec((1, _K), lambda ni: (0, 0)),
        ],
        out_specs=pl.BlockSpec((1, _BN, 1), lambda ni: (ni, 0, 0)),
        out_shape=jax.ShapeDtypeStruct((_NI, _BN, 1), jnp.int32),
    )(z, z2, embedding, e2)
    return out.reshape(_N)


_NC = 2
_NS = 16
_NW = _NC * _NS
_BW = _N // _NW


@functools.cache
def _make_sc_gather():
    @functools.partial(
        pl.kernel,
        mesh=plsc.VectorSubcoreMesh(core_axis_name="c", subcore_axis_name="s"),
        out_type=jax.ShapeDtypeStruct((_N, _D), jnp.float32),
        scratch_types=[
            pltpu.VMEM((_BW,), jnp.int32),
            pltpu.VMEM((_BW, _D), jnp.float32),
            pltpu.SemaphoreType.DMA,
        ],
    )
    def _sc_gather(table_hbm, idx_hbm, out_hbm, idx_v, rows_v, sem):
        wid = lax.axis_index("s") * _NC + lax.axis_index("c")
        base = wid * _BW
        pltpu.sync_copy(idx_hbm.at[pl.ds(base, _BW)], idx_v)
        pltpu.async_copy(table_hbm.at[idx_v], rows_v, sem).wait()
        pltpu.sync_copy(rows_v, out_hbm.at[pl.ds(base, _BW)])

    return _sc_gather


def _finish_body(z_ref, zq_ref, out_ref, ssq_ref):
    d = zq_ref[...] - z_ref[...]
    out_ref[...] = z_ref[...] + d
    ssq_ref[...] = jnp.sum(d * d).reshape(1, 1, 1)


def _finish(z, z_q):
    return pl.pallas_call(
        _finish_body,
        grid=(_NI,),
        in_specs=[
            pl.BlockSpec((_BN, _D), lambda ni: (ni, 0)),
            pl.BlockSpec((_BN, _D), lambda ni: (ni, 0)),
        ],
        out_specs=[
            pl.BlockSpec((_BN, _D), lambda ni: (ni, 0)),
            pl.BlockSpec((1, 1, 1), lambda ni: (ni, 0, 0)),
        ],
        out_shape=[
            jax.ShapeDtypeStruct((_N, _D), jnp.float32),
            jax.ShapeDtypeStruct((_NI, 1, 1), jnp.float32),
        ],
    )(z, z_q)


def kernel(z_e, embedding):
    z = z_e.reshape(-1, _D)
    z2 = jnp.sum(z * z, axis=1, keepdims=True)
    e2 = jnp.sum(embedding * embedding, axis=1)[None, :]
    indices = _argmin_indices(z, z2, e2, embedding)
    z_q = _make_sc_gather()(embedding, indices)
    z_q_st, ssq = _finish(z, z_q)
    m = jnp.sum(ssq) / (_N * _D)
    loss = m + _COMMIT * m
    return z_q_st.reshape(z_e.shape), loss
